# baseline (speedup 1.0000x reference)
import functools

import jax
import jax.numpy as jnp
from jax.experimental import pallas as pl
from jax.experimental.pallas import tpu as pltpu

_R = 512
_K = 4


def _mlp_body(c_ref, n_ref, w1_ref, w2_ref, m_ref, o_ref, *, rows, S, D):
    c = c_ref[...]
    x = n_ref[...]
    w1 = w1_ref[...]
    w2 = w2_ref[...]
    sn = jnp.tanh(jax.lax.dot(x, w1, preferred_element_type=jnp.float32))
    pn = jnp.sum((sn * x).reshape(rows, S, D), axis=1)
    sc = jnp.tanh(jax.lax.dot(c, w1, preferred_element_type=jnp.float32))
    t = pn + sc * c
    agg = jax.lax.dot(t, w2, preferred_element_type=jnp.float32)
    m = m_ref[...]
    o_ref[...] = jnp.where(m > 0.0, c, agg)


def _mlp_chunk(center_feat, neighbor_flat, W1, W2, is_neg, *, interpret=False):
    Bc, D = center_feat.shape
    S = neighbor_flat.shape[0] // Bc
    R = min(_R, Bc)
    grid = (Bc // R,)
    body = functools.partial(_mlp_body, rows=R, S=S, D=D)
    return pl.pallas_call(
        body,
        grid=grid,
        in_specs=[
            pl.BlockSpec((R, D), lambda i: (i, 0)),
            pl.BlockSpec((R * S, D), lambda i: (i, 0)),
            pl.BlockSpec((D, D), lambda i: (0, 0)),
            pl.BlockSpec((D, D), lambda i: (0, 0)),
            pl.BlockSpec((R, 1), lambda i: (i, 0)),
        ],
        out_specs=pl.BlockSpec((R, D), lambda i: (i, 0)),
        out_shape=jax.ShapeDtypeStruct((Bc, D), jnp.float32),
        compiler_params=pltpu.CompilerParams(
            dimension_semantics=("parallel",),
        ),
        interpret=interpret,
    )(center_feat, neighbor_flat, W1, W2, is_neg)


def kernel(center_feat, neighbor_feats, W1, W2):
    B, D = center_feat.shape
    S = neighbor_feats.shape[1]
    ano = int(B * 0.1)
    batch_center = jnp.mean(center_feat, axis=-1)
    diff_center = jnp.sum(center_feat - batch_center[:, None], axis=-1)
    sorted_idx = jnp.argsort(diff_center)
    neg_idx = sorted_idx[B - ano:]
    is_neg = jnp.zeros((B,), jnp.float32).at[neg_idx].set(1.0)[:, None]
    Bc = B // _K
    outs = []
    for k in range(_K):
        sl = slice(k * Bc, (k + 1) * Bc)
        outs.append(_mlp_chunk(
            center_feat[sl],
            neighbor_feats[sl].reshape(Bc * S, D),
            W1, W2, is_neg[sl]))
    out = jnp.concatenate(outs, axis=0)
    return out, neg_idx


# speedup vs baseline: 1.2340x; 1.2340x over previous
import functools

import jax
import jax.numpy as jnp
from jax.experimental import pallas as pl
from jax.experimental.pallas import tpu as pltpu

_R = 512


def _mlp_body(c_ref, n_ref, w1_ref, w1p_ref, w2_ref, m_ref, o_ref, *,
              rows, S, D):
    c = c_ref[...]
    x2 = n_ref[...]
    w1p = w1p_ref[...]
    sn = jnp.tanh(jax.lax.dot(x2, w1p, preferred_element_type=jnp.float32))
    p = (sn * x2).reshape(rows, S // 2, 2 * D)
    ps = jnp.sum(p, axis=1)
    pn = ps[:, :D] + ps[:, D:]
    w1 = w1_ref[...]
    sc = jnp.tanh(jax.lax.dot(c, w1, preferred_element_type=jnp.float32))
    t = pn + sc * c
    agg = jax.lax.dot(t, w2_ref[...], preferred_element_type=jnp.float32)
    m = m_ref[...]
    o_ref[...] = jnp.where(m > 0.0, c, agg)


def _mlp_all_rows(center_feat, neighbor_packed, W1, W1p, W2, is_neg, *,
                  interpret=False):
    B, D = center_feat.shape
    S = 2 * neighbor_packed.shape[0] // B
    R = _R
    grid = (B // R,)
    body = functools.partial(_mlp_body, rows=R, S=S, D=D)
    return pl.pallas_call(
        body,
        grid=grid,
        in_specs=[
            pl.BlockSpec((R, D), lambda i: (i, 0)),
            pl.BlockSpec((R * S // 2, 2 * D), lambda i: (i, 0)),
            pl.BlockSpec((D, D), lambda i: (0, 0)),
            pl.BlockSpec((2 * D, 2 * D), lambda i: (0, 0)),
            pl.BlockSpec((D, D), lambda i: (0, 0)),
            pl.BlockSpec((R, 1), lambda i: (i, 0)),
        ],
        out_specs=pl.BlockSpec((R, D), lambda i: (i, 0)),
        out_shape=jax.ShapeDtypeStruct((B, D), jnp.float32),
        compiler_params=pltpu.CompilerParams(
            dimension_semantics=("parallel",),
        ),
        interpret=interpret,
    )(center_feat, neighbor_packed, W1, W1p, W2, is_neg)


def kernel(center_feat, neighbor_feats, W1, W2):
    B, D = center_feat.shape
    S = neighbor_feats.shape[1]
    ano = int(B * 0.1)
    batch_center = jnp.mean(center_feat, axis=-1)
    diff_center = jnp.sum(center_feat - batch_center[:, None], axis=-1)
    sorted_idx = jnp.argsort(diff_center)
    neg_idx = sorted_idx[B - ano:]
    is_neg = jnp.zeros((B,), jnp.float32).at[neg_idx].set(1.0)[:, None]
    W1p = jnp.zeros((2 * D, 2 * D), jnp.float32)
    W1p = W1p.at[:D, :D].set(W1).at[D:, D:].set(W1)
    out = _mlp_all_rows(center_feat,
                        neighbor_feats.reshape(B * S // 2, 2 * D),
                        W1, W1p, W2, is_neg)
    return out, neg_idx


# speedup vs baseline: 2.1701x; 1.7586x over previous
import functools

import jax
import jax.numpy as jnp
from jax.experimental import pallas as pl
from jax.experimental.pallas import tpu as pltpu

_R = 512


def _mlp_body(c_ref, n_ref, w1_ref, w1b_ref, w2_ref, m_ref, o_ref, *,
              rows, S, D):
    c = c_ref[...]
    x = n_ref[...]
    w1b = w1b_ref[...]
    sn = jnp.tanh(jax.lax.dot(x, w1b, preferred_element_type=jnp.float32))
    xf = x.astype(jnp.float32)
    pn = jnp.sum((sn * xf).reshape(rows, S, D), axis=1)
    w1 = w1_ref[...]
    sc = jnp.tanh(jax.lax.dot(c, w1, preferred_element_type=jnp.float32))
    t = pn + sc * c
    agg = jax.lax.dot(t, w2_ref[...], preferred_element_type=jnp.float32)
    m = m_ref[...]
    o_ref[...] = jnp.where(m > 0.0, c, agg)


def _mlp_all_rows(center_feat, neighbor_flat, W1, W1b, W2, is_neg, *,
                  interpret=False):
    B, D = center_feat.shape
    S = neighbor_flat.shape[0] // B
    R = _R
    body = functools.partial(_mlp_body, rows=R, S=S, D=D)
    return pl.pallas_call(
        body,
        grid=(B // R,),
        in_specs=[
            pl.BlockSpec((R, D), lambda i: (i, 0)),
            pl.BlockSpec((R * S, D), lambda i: (i, 0)),
            pl.BlockSpec((D, D), lambda i: (0, 0)),
            pl.BlockSpec((D, D), lambda i: (0, 0)),
            pl.BlockSpec((D, D), lambda i: (0, 0)),
            pl.BlockSpec((R, 1), lambda i: (i, 0)),
        ],
        out_specs=pl.BlockSpec((R, D), lambda i: (i, 0)),
        out_shape=jax.ShapeDtypeStruct((B, D), jnp.float32),
        compiler_params=pltpu.CompilerParams(
            dimension_semantics=("parallel",),
        ),
        interpret=interpret,
    )(center_feat, neighbor_flat, W1, W1b, W2, is_neg)


def kernel(center_feat, neighbor_feats, W1, W2):
    B, D = center_feat.shape
    S = neighbor_feats.shape[1]
    ano = int(B * 0.1)
    batch_center = jnp.mean(center_feat, axis=-1)
    diff_center = jnp.sum(center_feat - batch_center[:, None], axis=-1)
    sorted_idx = jnp.argsort(diff_center)
    neg_idx = sorted_idx[B - ano:]
    is_neg = jnp.zeros((B,), jnp.float32).at[neg_idx].set(1.0)[:, None]
    flat = neighbor_feats.reshape(B * S, D).astype(jnp.bfloat16)
    out = _mlp_all_rows(center_feat, flat, W1, W1.astype(jnp.bfloat16), W2,
                        is_neg)
    return out, neg_idx


# speedup vs baseline: 2.2621x; 1.0424x over previous
import functools

import jax
import jax.numpy as jnp
from jax.experimental import pallas as pl
from jax.experimental.pallas import tpu as pltpu

_R = 512


def _mlp_body(c_ref, n_ref, w1_ref, w1p_ref, w2_ref, m_ref, o_ref, *,
              rows, S, D):
    c = c_ref[...]
    w1p = w1p_ref[...]
    nchunks = S * D // (2 * D)
    acc = jnp.zeros((rows, 2 * D), jnp.float32)
    for j in range(nchunks):
        xj = n_ref[:, 2 * D * j:2 * D * (j + 1)]
        sn = jnp.tanh(jax.lax.dot(xj, w1p,
                                  preferred_element_type=jnp.float32))
        acc = acc + sn * xj
    pn = acc[:, :D] + acc[:, D:]
    w1 = w1_ref[...]
    sc = jnp.tanh(jax.lax.dot(c, w1, preferred_element_type=jnp.float32))
    t = pn + sc * c
    agg = jax.lax.dot(t, w2_ref[...], preferred_element_type=jnp.float32)
    m = m_ref[...]
    o_ref[...] = jnp.where(m > 0.0, c, agg)


def _mlp_all_rows(center_feat, neighbor_rows, W1, W1p, W2, is_neg, *,
                  interpret=False):
    B, D = center_feat.shape
    SD = neighbor_rows.shape[1]
    S = SD // D
    R = _R
    body = functools.partial(_mlp_body, rows=R, S=S, D=D)
    return pl.pallas_call(
        body,
        grid=(B // R,),
        in_specs=[
            pl.BlockSpec((R, D), lambda i: (i, 0)),
            pl.BlockSpec((R, SD), lambda i: (i, 0)),
            pl.BlockSpec((D, D), lambda i: (0, 0)),
            pl.BlockSpec((2 * D, 2 * D), lambda i: (0, 0)),
            pl.BlockSpec((D, D), lambda i: (0, 0)),
            pl.BlockSpec((R, 1), lambda i: (i, 0)),
        ],
        out_specs=pl.BlockSpec((R, D), lambda i: (i, 0)),
        out_shape=jax.ShapeDtypeStruct((B, D), jnp.float32),
        compiler_params=pltpu.CompilerParams(
            dimension_semantics=("parallel",),
        ),
        interpret=interpret,
    )(center_feat, neighbor_rows, W1, W1p, W2, is_neg)


def kernel(center_feat, neighbor_feats, W1, W2):
    B, D = center_feat.shape
    S = neighbor_feats.shape[1]
    ano = int(B * 0.1)
    batch_center = jnp.mean(center_feat, axis=-1)
    diff_center = jnp.sum(center_feat - batch_center[:, None], axis=-1)
    sorted_idx = jnp.argsort(diff_center)
    neg_idx = sorted_idx[B - ano:]
    is_neg = jnp.zeros((B,), jnp.float32).at[neg_idx].set(1.0)[:, None]
    W1p = jnp.zeros((2 * D, 2 * D), jnp.float32)
    W1p = W1p.at[:D, :D].set(W1).at[D:, D:].set(W1)
    out = _mlp_all_rows(center_feat, neighbor_feats.reshape(B, S * D),
                        W1, W1p, W2, is_neg)
    return out, neg_idx
